# baseline (speedup 1.0000x reference)
import functools

import jax
import jax.numpy as jnp
from jax import lax
from jax.experimental import pallas as pl
from jax.experimental.pallas import tpu as pltpu
from jax.experimental.pallas import tpu_sc as plsc

E = 320000
N_NODES = 10000
D = 128
L = 16
NC = 2
NS = 16
NW = NC * NS
EPW = E // NW
C = 64
NCHUNK = EPW // C
TAIL = EPW - NCHUNK * C
NPAIR = NCHUNK // 2

_mesh = plsc.VectorSubcoreMesh(core_axis_name="c", subcore_axis_name="s")


@functools.partial(
    pl.kernel,
    out_type=jax.ShapeDtypeStruct((E,), jnp.float32),
    mesh=_mesh,
    scratch_types=[
        pltpu.VMEM_SHARED((N_NODES, D // 2), jnp.int32),
        pltpu.VMEM((C,), jnp.int32),
        pltpu.VMEM((C,), jnp.int32),
        pltpu.VMEM((C,), jnp.int32),
        pltpu.VMEM((C,), jnp.int32),
        pltpu.VMEM((C, D // 2), jnp.int32),
        pltpu.VMEM((C, D // 2), jnp.int32),
        pltpu.VMEM((C, D // 2), jnp.int32),
        pltpu.VMEM((C, D // 2), jnp.int32),
        pltpu.VMEM((EPW,), jnp.float32),
        pltpu.VMEM((L * L,), jnp.float32),
        pltpu.SemaphoreType.DMA,
        pltpu.SemaphoreType.DMA,
        pltpu.SemaphoreType.DMA,
        pltpu.SemaphoreType.DMA,
        pltpu.SemaphoreType.DMA,
        pltpu.SemaphoreType.DMA,
        pltpu.SemaphoreType.DMA,
        pltpu.SemaphoreType.DMA,
    ],
    compiler_params=pltpu.CompilerParams(needs_layout_passes=False, use_tc_tiling_on_sc=False),
)
def _decode(z_hbm, src_hbm, dst_hbm, out_hbm,
            z_sp, sidx_a, didx_a, sidx_b, didx_b,
            srows_a, drows_a, srows_b, drows_b, oval, tstage,
            sem_sa, sem_da, sem_sb, sem_db,
            sem_isa, sem_ida, sem_isb, sem_idb):
    wid = lax.axis_index("s") * NC + lax.axis_index("c")
    sid = lax.axis_index("s")
    base = wid * EPW
    rps = (N_NODES // NS) // 8 * 8
    pltpu.sync_copy(z_hbm.at[pl.ds(sid * rps, rps)],
                    z_sp.at[pl.ds(sid * rps, rps)])

    @pl.when(sid == 0)
    def _copy_z_tail():
        pltpu.sync_copy(z_hbm.at[pl.ds(NS * rps, N_NODES - NS * rps)],
                        z_sp.at[pl.ds(NS * rps, N_NODES - NS * rps)])

    plsc.subcore_barrier()

    def start_idx(ci, si, di, sem_s, sem_d):
        pltpu.async_copy(src_hbm.at[pl.ds(base + ci * C, C)], si, sem_s)
        pltpu.async_copy(dst_hbm.at[pl.ds(base + ci * C, C)], di, sem_d)

    def wait_idx(si, di, sem_s, sem_d):
        pltpu.make_async_copy(src_hbm.at[pl.ds(base, C)], si, sem_s).wait()
        pltpu.make_async_copy(dst_hbm.at[pl.ds(base, C)], di, sem_d).wait()

    def start_rows(zsrc, si, di, srows, drows, sem_s, sem_d):
        pltpu.async_copy(zsrc.at[si], srows, sem_s)
        pltpu.async_copy(zsrc.at[di], drows, sem_d)

    def wait_rows(zsrc, si, di, srows, drows, sem_s, sem_d):
        pltpu.make_async_copy(zsrc.at[si], srows, sem_s).wait()
        pltpu.make_async_copy(zsrc.at[di], drows, sem_d).wait()

    def edge_dot(srows, drows, row):
        acc = None
        for k in range(D // (2 * L)):
            sw = plsc.bitcast(srows[row, pl.ds(k * L, L)], jnp.bfloat16)
            dw = plsc.bitcast(drows[row, pl.ds(k * L, L)], jnp.bfloat16)
            sa, sb = plsc.unpack(sw, format=plsc.PackFormat.INTERLEAVED)
            da, db = plsc.unpack(dw, format=plsc.PackFormat.INTERLEAVED)
            t = sa * da + sb * db
            acc = t if acc is None else acc + t
        return acc

    def compute(ci, srows, drows):
        def group_body(g, carry2):
            lanes = lax.iota(jnp.int32, L)
            for e in range(L):
                plsc.store_scatter(tstage, [lanes * L + e],
                                   edge_dot(srows, drows, g * L + e))
            dots = tstage[pl.ds(0, L)]
            for r in range(1, L):
                dots = dots + tstage[pl.ds(r * L, L)]
            oval[pl.ds(ci * C + g * L, L)] = 1.0 / (1.0 + jnp.exp(-dots))
            return carry2

        lax.fori_loop(0, C // L, group_body, 0)

    pltpu.sync_copy(src_hbm.at[pl.ds(base, C)], sidx_a)
    pltpu.sync_copy(dst_hbm.at[pl.ds(base, C)], didx_a)
    start_idx(1, sidx_b, didx_b, sem_isb, sem_idb)
    start_rows(z_sp, sidx_a, didx_a, srows_a, drows_a, sem_sa, sem_da)

    def pair_body(p, carry):
        c0 = 2 * p
        wait_idx(sidx_b, didx_b, sem_isb, sem_idb)
        start_rows(z_hbm, sidx_b, didx_b, srows_b, drows_b, sem_sb, sem_db)
        wait_rows(z_sp, sidx_a, didx_a, srows_a, drows_a, sem_sa, sem_da)

        @pl.when(c0 + 2 < NCHUNK)
        def _():
            start_idx(c0 + 2, sidx_a, didx_a, sem_isa, sem_ida)

        compute(c0, srows_a, drows_a)

        @pl.when(c0 + 2 < NCHUNK)
        def _():
            wait_idx(sidx_a, didx_a, sem_isa, sem_ida)
            start_rows(z_sp, sidx_a, didx_a, srows_a, drows_a,
                       sem_sa, sem_da)

        wait_rows(z_hbm, sidx_b, didx_b, srows_b, drows_b, sem_sb, sem_db)

        @pl.when(c0 + 3 < NCHUNK)
        def _():
            start_idx(c0 + 3, sidx_b, didx_b, sem_isb, sem_idb)

        compute(c0 + 1, srows_b, drows_b)
        return carry

    lax.fori_loop(0, NPAIR, pair_body, 0)

    pltpu.sync_copy(src_hbm.at[pl.ds(base + NCHUNK * C, TAIL)],
                    sidx_a.at[pl.ds(0, TAIL)])
    pltpu.sync_copy(dst_hbm.at[pl.ds(base + NCHUNK * C, TAIL)],
                    didx_a.at[pl.ds(0, TAIL)])
    pltpu.async_copy(z_sp.at[sidx_a.at[pl.ds(0, TAIL)]],
                     srows_a.at[pl.ds(0, TAIL)], sem_sa)
    pltpu.async_copy(z_sp.at[didx_a.at[pl.ds(0, TAIL)]],
                     drows_a.at[pl.ds(0, TAIL)], sem_da)
    pltpu.make_async_copy(z_sp.at[sidx_a.at[pl.ds(0, TAIL)]],
                          srows_a.at[pl.ds(0, TAIL)], sem_sa).wait()
    pltpu.make_async_copy(z_sp.at[didx_a.at[pl.ds(0, TAIL)]],
                          drows_a.at[pl.ds(0, TAIL)], sem_da).wait()
    lanes = lax.iota(jnp.int32, L)
    for e in range(L):
        plsc.store_scatter(tstage, [lanes * L + e],
                           edge_dot(srows_a, drows_a, e))
    dots = tstage[pl.ds(0, L)]
    for r in range(1, L):
        dots = dots + tstage[pl.ds(r * L, L)]
    oval[pl.ds(NCHUNK * C, L)] = 1.0 / (1.0 + jnp.exp(-dots))

    pltpu.sync_copy(oval, out_hbm.at[pl.ds(base, EPW)])


def kernel(z, edge_index):
    ei = edge_index.astype(jnp.int32)
    zb = z.astype(jnp.bfloat16).reshape(N_NODES, D // 2, 2)
    zi = lax.bitcast_convert_type(zb, jnp.int32)
    return _decode(zi, ei[0], ei[1])


# speedup vs baseline: 1.0035x; 1.0035x over previous
import functools

import jax
import jax.numpy as jnp
from jax import lax
from jax.experimental import pallas as pl
from jax.experimental.pallas import tpu as pltpu
from jax.experimental.pallas import tpu_sc as plsc

E = 320000
N_NODES = 10000
D = 128
L = 16
NC = 2
NS = 16
NW = NC * NS
EPW = E // NW
C = 64
NCHUNK = EPW // C
TAIL = EPW - NCHUNK * C
NPAIR = NCHUNK // 2

_mesh = plsc.VectorSubcoreMesh(core_axis_name="c", subcore_axis_name="s")


@functools.partial(
    pl.kernel,
    out_type=jax.ShapeDtypeStruct((E,), jnp.float32),
    mesh=_mesh,
    scratch_types=[
        pltpu.VMEM_SHARED((N_NODES, D // 2), jnp.int32),
        pltpu.VMEM((C,), jnp.int32),
        pltpu.VMEM((C,), jnp.int32),
        pltpu.VMEM((C,), jnp.int32),
        pltpu.VMEM((C,), jnp.int32),
        pltpu.VMEM((C, D // 2), jnp.int32),
        pltpu.VMEM((C, D // 2), jnp.int32),
        pltpu.VMEM((C, D // 2), jnp.int32),
        pltpu.VMEM((C, D // 2), jnp.int32),
        pltpu.VMEM((EPW,), jnp.float32),
        pltpu.VMEM((L * L,), jnp.float32),
        pltpu.SemaphoreType.DMA,
        pltpu.SemaphoreType.DMA,
        pltpu.SemaphoreType.DMA,
        pltpu.SemaphoreType.DMA,
        pltpu.SemaphoreType.DMA,
        pltpu.SemaphoreType.DMA,
        pltpu.SemaphoreType.DMA,
        pltpu.SemaphoreType.DMA,
    ],
    compiler_params=pltpu.CompilerParams(needs_layout_passes=False, use_tc_tiling_on_sc=False),
)
def _decode(z_hbm, src_hbm, dst_hbm, out_hbm,
            z_sp, sidx_a, didx_a, sidx_b, didx_b,
            srows_a, drows_a, srows_b, drows_b, oval, tstage,
            sem_sa, sem_da, sem_sb, sem_db,
            sem_isa, sem_ida, sem_isb, sem_idb):
    wid = lax.axis_index("s") * NC + lax.axis_index("c")
    sid = lax.axis_index("s")
    base = wid * EPW
    rps = (N_NODES // NS) // 8 * 8
    pltpu.sync_copy(z_hbm.at[pl.ds(sid * rps, rps)],
                    z_sp.at[pl.ds(sid * rps, rps)])

    @pl.when(sid == 0)
    def _copy_z_tail():
        pltpu.sync_copy(z_hbm.at[pl.ds(NS * rps, N_NODES - NS * rps)],
                        z_sp.at[pl.ds(NS * rps, N_NODES - NS * rps)])

    plsc.subcore_barrier()

    def start_idx(ci, si, di, sem_s, sem_d):
        pltpu.async_copy(src_hbm.at[pl.ds(base + ci * C, C)], si, sem_s)
        pltpu.async_copy(dst_hbm.at[pl.ds(base + ci * C, C)], di, sem_d)

    def wait_idx(si, di, sem_s, sem_d):
        pltpu.make_async_copy(src_hbm.at[pl.ds(base, C)], si, sem_s).wait()
        pltpu.make_async_copy(dst_hbm.at[pl.ds(base, C)], di, sem_d).wait()

    def start_rows(zsrc, si, di, srows, drows, sem_s, sem_d):
        pltpu.async_copy(zsrc.at[si], srows, sem_s)
        pltpu.async_copy(zsrc.at[di], drows, sem_d)

    def wait_rows(zsrc, si, di, srows, drows, sem_s, sem_d):
        pltpu.make_async_copy(zsrc.at[si], srows, sem_s).wait()
        pltpu.make_async_copy(zsrc.at[di], drows, sem_d).wait()

    def edge_dot(srows, drows, row):
        mask = jnp.full((L,), -65536, jnp.int32)
        acc = None
        for k in range(D // (2 * L)):
            sw = srows[row, pl.ds(k * L, L)]
            dw = drows[row, pl.ds(k * L, L)]
            s_lo = plsc.bitcast(lax.shift_left(sw, 16), jnp.float32)
            d_lo = plsc.bitcast(lax.shift_left(dw, 16), jnp.float32)
            s_hi = plsc.bitcast(sw & mask, jnp.float32)
            d_hi = plsc.bitcast(dw & mask, jnp.float32)
            t = s_lo * d_lo + s_hi * d_hi
            acc = t if acc is None else acc + t
        return acc

    def compute(ci, srows, drows):
        def group_body(g, carry2):
            lanes = lax.iota(jnp.int32, L)
            for e in range(L):
                plsc.store_scatter(tstage, [lanes * L + e],
                                   edge_dot(srows, drows, g * L + e))
            dots = tstage[pl.ds(0, L)]
            for r in range(1, L):
                dots = dots + tstage[pl.ds(r * L, L)]
            oval[pl.ds(ci * C + g * L, L)] = 1.0 / (1.0 + jnp.exp(-dots))
            return carry2

        lax.fori_loop(0, C // L, group_body, 0)

    pltpu.sync_copy(src_hbm.at[pl.ds(base, C)], sidx_a)
    pltpu.sync_copy(dst_hbm.at[pl.ds(base, C)], didx_a)
    start_idx(1, sidx_b, didx_b, sem_isb, sem_idb)
    start_rows(z_sp, sidx_a, didx_a, srows_a, drows_a, sem_sa, sem_da)

    def pair_body(p, carry):
        c0 = 2 * p
        wait_idx(sidx_b, didx_b, sem_isb, sem_idb)
        start_rows(z_sp, sidx_b, didx_b, srows_b, drows_b, sem_sb, sem_db)
        wait_rows(z_sp, sidx_a, didx_a, srows_a, drows_a, sem_sa, sem_da)

        @pl.when(c0 + 2 < NCHUNK)
        def _():
            start_idx(c0 + 2, sidx_a, didx_a, sem_isa, sem_ida)

        compute(c0, srows_a, drows_a)

        @pl.when(c0 + 2 < NCHUNK)
        def _():
            wait_idx(sidx_a, didx_a, sem_isa, sem_ida)
            start_rows(z_sp, sidx_a, didx_a, srows_a, drows_a,
                       sem_sa, sem_da)

        wait_rows(z_sp, sidx_b, didx_b, srows_b, drows_b, sem_sb, sem_db)

        @pl.when(c0 + 3 < NCHUNK)
        def _():
            start_idx(c0 + 3, sidx_b, didx_b, sem_isb, sem_idb)

        compute(c0 + 1, srows_b, drows_b)
        return carry

    lax.fori_loop(0, NPAIR, pair_body, 0)

    pltpu.sync_copy(src_hbm.at[pl.ds(base + NCHUNK * C, TAIL)],
                    sidx_a.at[pl.ds(0, TAIL)])
    pltpu.sync_copy(dst_hbm.at[pl.ds(base + NCHUNK * C, TAIL)],
                    didx_a.at[pl.ds(0, TAIL)])
    pltpu.async_copy(z_sp.at[sidx_a.at[pl.ds(0, TAIL)]],
                     srows_a.at[pl.ds(0, TAIL)], sem_sa)
    pltpu.async_copy(z_sp.at[didx_a.at[pl.ds(0, TAIL)]],
                     drows_a.at[pl.ds(0, TAIL)], sem_da)
    pltpu.make_async_copy(z_sp.at[sidx_a.at[pl.ds(0, TAIL)]],
                          srows_a.at[pl.ds(0, TAIL)], sem_sa).wait()
    pltpu.make_async_copy(z_sp.at[didx_a.at[pl.ds(0, TAIL)]],
                          drows_a.at[pl.ds(0, TAIL)], sem_da).wait()
    lanes = lax.iota(jnp.int32, L)
    for e in range(L):
        plsc.store_scatter(tstage, [lanes * L + e],
                           edge_dot(srows_a, drows_a, e))
    dots = tstage[pl.ds(0, L)]
    for r in range(1, L):
        dots = dots + tstage[pl.ds(r * L, L)]
    oval[pl.ds(NCHUNK * C, L)] = 1.0 / (1.0 + jnp.exp(-dots))

    pltpu.sync_copy(oval, out_hbm.at[pl.ds(base, EPW)])


def kernel(z, edge_index):
    ei = edge_index.astype(jnp.int32)
    zb = z.astype(jnp.bfloat16).reshape(N_NODES, D // 2, 2)
    zi = lax.bitcast_convert_type(zb, jnp.int32)
    return _decode(zi, ei[0], ei[1])


# speedup vs baseline: 1.1185x; 1.1145x over previous
import functools

import jax
import jax.numpy as jnp
from jax import lax
from jax.experimental import pallas as pl
from jax.experimental.pallas import tpu as pltpu
from jax.experimental.pallas import tpu_sc as plsc

E = 320000
N_NODES = 10000
D = 128
L = 16
NC = 2
NS = 16
NW = NC * NS
EPW = E // NW
C = 64
NCHUNK = EPW // C
TAIL = EPW - NCHUNK * C
NPAIR = NCHUNK // 2

_mesh = plsc.VectorSubcoreMesh(core_axis_name="c", subcore_axis_name="s")


@functools.partial(
    pl.kernel,
    out_type=jax.ShapeDtypeStruct((E,), jnp.float32),
    mesh=_mesh,
    scratch_types=[
        pltpu.VMEM_SHARED((N_NODES, D), jnp.float32),
        pltpu.VMEM((C,), jnp.int32),
        pltpu.VMEM((C,), jnp.int32),
        pltpu.VMEM((C,), jnp.int32),
        pltpu.VMEM((C,), jnp.int32),
        pltpu.VMEM((C, D), jnp.float32),
        pltpu.VMEM((C, D), jnp.float32),
        pltpu.VMEM((C, D), jnp.float32),
        pltpu.VMEM((C, D), jnp.float32),
        pltpu.VMEM((EPW,), jnp.float32),
        pltpu.VMEM((L * L,), jnp.float32),
        pltpu.SemaphoreType.DMA,
        pltpu.SemaphoreType.DMA,
        pltpu.SemaphoreType.DMA,
        pltpu.SemaphoreType.DMA,
        pltpu.SemaphoreType.DMA,
        pltpu.SemaphoreType.DMA,
        pltpu.SemaphoreType.DMA,
        pltpu.SemaphoreType.DMA,
    ],
    compiler_params=pltpu.CompilerParams(needs_layout_passes=False),
)
def _decode(z_hbm, src_hbm, dst_hbm, out_hbm,
            z_sp, sidx_a, didx_a, sidx_b, didx_b,
            srows_a, drows_a, srows_b, drows_b, oval, tstage,
            sem_sa, sem_da, sem_sb, sem_db,
            sem_isa, sem_ida, sem_isb, sem_idb):
    wid = lax.axis_index("s") * NC + lax.axis_index("c")
    sid = lax.axis_index("s")
    base = wid * EPW
    rps = (N_NODES // NS) // 8 * 8
    pltpu.sync_copy(z_hbm.at[pl.ds(sid * rps, rps)],
                    z_sp.at[pl.ds(sid * rps, rps)])

    @pl.when(sid == 0)
    def _copy_z_tail():
        pltpu.sync_copy(z_hbm.at[pl.ds(NS * rps, N_NODES - NS * rps)],
                        z_sp.at[pl.ds(NS * rps, N_NODES - NS * rps)])

    plsc.subcore_barrier()

    def start_idx(ci, si, di, sem_s, sem_d):
        pltpu.async_copy(src_hbm.at[pl.ds(base + ci * C, C)], si, sem_s)
        pltpu.async_copy(dst_hbm.at[pl.ds(base + ci * C, C)], di, sem_d)

    def wait_idx(si, di, sem_s, sem_d):
        pltpu.make_async_copy(src_hbm.at[pl.ds(base, C)], si, sem_s).wait()
        pltpu.make_async_copy(dst_hbm.at[pl.ds(base, C)], di, sem_d).wait()

    def start_rows(si, di, srows, drows, sem_s, sem_d):
        pltpu.async_copy(z_sp.at[si], srows, sem_s)
        pltpu.async_copy(z_sp.at[di], drows, sem_d)

    def wait_rows(si, di, srows, drows, sem_s, sem_d):
        pltpu.make_async_copy(z_sp.at[si], srows, sem_s).wait()
        pltpu.make_async_copy(z_sp.at[di], drows, sem_d).wait()

    def compute(ci, srows, drows):
        def group_body(g, carry2):
            lanes = lax.iota(jnp.int32, L)
            for e in range(L):
                row = g * L + e
                acc = srows[row, pl.ds(0, L)] * drows[row, pl.ds(0, L)]
                for k in range(1, D // L):
                    acc = acc + (srows[row, pl.ds(k * L, L)]
                                 * drows[row, pl.ds(k * L, L)])
                plsc.store_scatter(tstage, [lanes * L + e], acc)
            dots = tstage[pl.ds(0, L)]
            for r in range(1, L):
                dots = dots + tstage[pl.ds(r * L, L)]
            oval[pl.ds(ci * C + g * L, L)] = 1.0 / (1.0 + jnp.exp(-dots))
            return carry2

        lax.fori_loop(0, C // L, group_body, 0)

    pltpu.sync_copy(src_hbm.at[pl.ds(base, C)], sidx_a)
    pltpu.sync_copy(dst_hbm.at[pl.ds(base, C)], didx_a)
    start_idx(1, sidx_b, didx_b, sem_isb, sem_idb)
    start_rows(sidx_a, didx_a, srows_a, drows_a, sem_sa, sem_da)

    def pair_body(p, carry):
        c0 = 2 * p
        wait_idx(sidx_b, didx_b, sem_isb, sem_idb)
        start_rows(sidx_b, didx_b, srows_b, drows_b, sem_sb, sem_db)
        wait_rows(sidx_a, didx_a, srows_a, drows_a, sem_sa, sem_da)

        @pl.when(c0 + 2 < NCHUNK)
        def _():
            start_idx(c0 + 2, sidx_a, didx_a, sem_isa, sem_ida)

        compute(c0, srows_a, drows_a)

        @pl.when(c0 + 2 < NCHUNK)
        def _():
            wait_idx(sidx_a, didx_a, sem_isa, sem_ida)
            start_rows(sidx_a, didx_a, srows_a, drows_a, sem_sa, sem_da)

        wait_rows(sidx_b, didx_b, srows_b, drows_b, sem_sb, sem_db)

        @pl.when(c0 + 3 < NCHUNK)
        def _():
            start_idx(c0 + 3, sidx_b, didx_b, sem_isb, sem_idb)

        compute(c0 + 1, srows_b, drows_b)
        return carry

    lax.fori_loop(0, NPAIR, pair_body, 0)

    pltpu.sync_copy(src_hbm.at[pl.ds(base + NCHUNK * C, TAIL)],
                    sidx_a.at[pl.ds(0, TAIL)])
    pltpu.sync_copy(dst_hbm.at[pl.ds(base + NCHUNK * C, TAIL)],
                    didx_a.at[pl.ds(0, TAIL)])
    pltpu.async_copy(z_sp.at[sidx_a.at[pl.ds(0, TAIL)]],
                     srows_a.at[pl.ds(0, TAIL)], sem_sa)
    pltpu.async_copy(z_sp.at[didx_a.at[pl.ds(0, TAIL)]],
                     drows_a.at[pl.ds(0, TAIL)], sem_da)
    pltpu.make_async_copy(z_sp.at[sidx_a.at[pl.ds(0, TAIL)]],
                          srows_a.at[pl.ds(0, TAIL)], sem_sa).wait()
    pltpu.make_async_copy(z_sp.at[didx_a.at[pl.ds(0, TAIL)]],
                          drows_a.at[pl.ds(0, TAIL)], sem_da).wait()
    lanes = lax.iota(jnp.int32, L)
    for e in range(L):
        acc = srows_a[e, pl.ds(0, L)] * drows_a[e, pl.ds(0, L)]
        for k in range(1, D // L):
            acc = acc + (srows_a[e, pl.ds(k * L, L)]
                         * drows_a[e, pl.ds(k * L, L)])
        plsc.store_scatter(tstage, [lanes * L + e], acc)
    dots = tstage[pl.ds(0, L)]
    for r in range(1, L):
        dots = dots + tstage[pl.ds(r * L, L)]
    oval[pl.ds(NCHUNK * C, L)] = 1.0 / (1.0 + jnp.exp(-dots))

    pltpu.sync_copy(oval, out_hbm.at[pl.ds(base, EPW)])


def kernel(z, edge_index):
    ei = edge_index.astype(jnp.int32)
    return _decode(z, ei[0], ei[1])
